# Initial kernel scaffold; baseline (speedup 1.0000x reference)
import jax
import jax.numpy as jnp
from jax.experimental import pallas as pl


def kernel(x, table, pos_enc):
    raise NotImplementedError("write your pallas kernel here")


# speedup vs baseline: 1.2515x; 1.2515x over previous
import functools
import math

import jax
import jax.numpy as jnp
from jax import lax
from jax.experimental import pallas as pl
from jax.experimental.pallas import tpu as pltpu
from jax.experimental.pallas import tpu_sc as plsc

D_MODEL = 128
SEQ_LEN = 2048
BATCH = 4
N_TOK = SEQ_LEN * BATCH

_NC = 2
_NS = 16
_NW = _NC * _NS
_ROWS_PER_W = N_TOK // _NW
_IDX_CHUNK = 128
_N_CHUNKS = _ROWS_PER_W // _IDX_CHUNK
_POS_PER_W = SEQ_LEN // _NW
_LANES = 16
_CPR = D_MODEL // _LANES
_SCALE = math.sqrt(float(D_MODEL))


def _sc_body(x_hbm, pe_hbm, table_hbm, out_hbm, idx_v, rows_v, pe_v, sem):
    wid = lax.axis_index("s") * _NC + lax.axis_index("c")
    base = wid * _ROWS_PER_W

    pltpu.sync_copy(x_hbm.at[pl.ds(wid * _N_CHUNKS, _N_CHUNKS)], idx_v)

    copies = [
        pltpu.async_copy(
            table_hbm.at[idx_v.at[j]],
            rows_v.at[pl.ds(j * _IDX_CHUNK, _IDX_CHUNK)],
            sem,
        )
        for j in range(_N_CHUNKS)
    ]
    pltpu.sync_copy(pe_hbm.at[pl.ds(wid * _POS_PER_W, _POS_PER_W)], pe_v)
    for c in copies:
        c.wait()

    scale = jnp.float32(_SCALE)

    def body(p, carry):
        for c in range(_CPR):
            sl = pl.ds(c * _LANES, _LANES)
            pe_c = pe_v[p, sl]
            for b in range(BATCH):
                r = p * BATCH + b
                rows_v[r, sl] = rows_v[r, sl] * scale + pe_c
        return carry

    lax.fori_loop(0, _POS_PER_W, body, 0, unroll=False)

    pltpu.sync_copy(rows_v, out_hbm.at[pl.ds(base, _ROWS_PER_W)])


_sc_embed = functools.partial(
    pl.kernel,
    out_type=jax.ShapeDtypeStruct((N_TOK, D_MODEL), jnp.float32),
    mesh=plsc.VectorSubcoreMesh(core_axis_name="c", subcore_axis_name="s"),
    scratch_types=[
        pltpu.VMEM((_NW * _N_CHUNKS // _NW, _IDX_CHUNK), jnp.int32),
        pltpu.VMEM((_ROWS_PER_W, D_MODEL), jnp.float32),
        pltpu.VMEM((_POS_PER_W, D_MODEL), jnp.float32),
        pltpu.SemaphoreType.DMA,
    ],
)(_sc_body)


@jax.jit
def kernel(x, table, pos_enc):
    x_flat = jnp.reshape(x.astype(jnp.int32), (_NW * _N_CHUNKS, _IDX_CHUNK))
    pe = jnp.reshape(pos_enc[:SEQ_LEN], (SEQ_LEN, D_MODEL))
    out = _sc_embed(x_flat, pe, table)
    return jnp.reshape(out, (SEQ_LEN, BATCH, D_MODEL))


# speedup vs baseline: 1.2619x; 1.0083x over previous
import functools
import math

import jax
import jax.numpy as jnp
from jax import lax
from jax.experimental import pallas as pl
from jax.experimental.pallas import tpu as pltpu
from jax.experimental.pallas import tpu_sc as plsc

D_MODEL = 128
SEQ_LEN = 2048
BATCH = 4
N_TOK = SEQ_LEN * BATCH

_NC = 2
_NS = 16
_NW = _NC * _NS
_ROWS_PER_W = N_TOK // _NW
_N_CHUNKS = 4
_IDX_CHUNK = _ROWS_PER_W // _N_CHUNKS
_POS_PER_W = SEQ_LEN // _NW
_POS_PER_CHUNK = _POS_PER_W // _N_CHUNKS
_LANES = 16
_CPR = D_MODEL // _LANES
_SCALE = math.sqrt(float(D_MODEL))


def _sc_body(x_hbm, pe_hbm, table_hbm, out_hbm, idx_v, rows_v, pe_v, sems,
             pe_sem):
    wid = lax.axis_index("s") * _NC + lax.axis_index("c")
    base = wid * _ROWS_PER_W

    pltpu.sync_copy(x_hbm.at[pl.ds(wid * _N_CHUNKS, _N_CHUNKS)], idx_v)

    gathers = [
        pltpu.async_copy(
            table_hbm.at[idx_v.at[j]],
            rows_v.at[pl.ds(j * _IDX_CHUNK, _IDX_CHUNK)],
            sems[j],
        )
        for j in range(_N_CHUNKS)
    ]
    pe_copy = pltpu.async_copy(
        pe_hbm.at[pl.ds(wid * _POS_PER_W, _POS_PER_W)], pe_v, pe_sem)

    scale = jnp.float32(_SCALE)

    def make_body(j):
        def body(i, carry):
            p = j * _POS_PER_CHUNK + i
            for c in range(_CPR):
                sl = pl.ds(c * _LANES, _LANES)
                pe_c = pe_v[p, sl]
                for b in range(BATCH):
                    r = p * BATCH + b
                    rows_v[r, sl] = rows_v[r, sl] * scale + pe_c
            return carry
        return body

    pe_copy.wait()
    stores = []
    for j in range(_N_CHUNKS):
        gathers[j].wait()
        lax.fori_loop(0, _POS_PER_CHUNK, make_body(j), 0, unroll=False)
        stores.append(
            pltpu.async_copy(
                rows_v.at[pl.ds(j * _IDX_CHUNK, _IDX_CHUNK)],
                out_hbm.at[pl.ds(base + j * _IDX_CHUNK, _IDX_CHUNK)],
                sems[j],
            )
        )
    for s in stores:
        s.wait()


_sc_embed = functools.partial(
    pl.kernel,
    out_type=jax.ShapeDtypeStruct((N_TOK, D_MODEL), jnp.float32),
    mesh=plsc.VectorSubcoreMesh(core_axis_name="c", subcore_axis_name="s"),
    scratch_types=[
        pltpu.VMEM((_N_CHUNKS, _IDX_CHUNK), jnp.int32),
        pltpu.VMEM((_ROWS_PER_W, D_MODEL), jnp.float32),
        pltpu.VMEM((_POS_PER_W, D_MODEL), jnp.float32),
        [pltpu.SemaphoreType.DMA] * _N_CHUNKS,
        pltpu.SemaphoreType.DMA,
    ],
)(_sc_body)


@jax.jit
def kernel(x, table, pos_enc):
    x_flat = jnp.reshape(x.astype(jnp.int32), (_NW * _N_CHUNKS, _IDX_CHUNK))
    pe = jnp.reshape(pos_enc[:SEQ_LEN], (SEQ_LEN, D_MODEL))
    out = _sc_embed(x_flat, pe, table)
    return jnp.reshape(out, (SEQ_LEN, BATCH, D_MODEL))


# speedup vs baseline: 1.4199x; 1.1252x over previous
import functools
import math

import jax
import jax.numpy as jnp
from jax import lax
from jax.experimental import pallas as pl
from jax.experimental.pallas import tpu as pltpu
from jax.experimental.pallas import tpu_sc as plsc

D_MODEL = 128
SEQ_LEN = 2048
BATCH = 4
N_TOK = SEQ_LEN * BATCH

_NC = 2
_NS = 16
_NW = _NC * _NS
_ROWS_PER_W = N_TOK // _NW
_N_CHUNKS = 4
_IDX_CHUNK = _ROWS_PER_W // _N_CHUNKS
_POS_PER_W = SEQ_LEN // _NW
_POS_PER_CHUNK = _POS_PER_W // _N_CHUNKS
_LANES = 16
_CPR = D_MODEL // _LANES
_SCALE = math.sqrt(float(D_MODEL))


def _sc_body(x_hbm, pe_hbm, table_hbm, out_hbm, idx_v, rows_v, pe_v, sems,
             pe_sem):
    wid = lax.axis_index("s") * _NC + lax.axis_index("c")
    base = wid * _ROWS_PER_W

    pltpu.sync_copy(x_hbm.at[pl.ds(wid * _N_CHUNKS, _N_CHUNKS)], idx_v)

    pe_copy = pltpu.async_copy(
        pe_hbm.at[pl.ds(wid * _POS_PER_W, _POS_PER_W)], pe_v, pe_sem)
    gathers = [
        pltpu.async_copy(
            table_hbm.at[idx_v.at[j]],
            rows_v.at[pl.ds(j * _IDX_CHUNK, _IDX_CHUNK)],
            sems[j],
        )
        for j in range(_N_CHUNKS)
    ]

    scale = jnp.float32(_SCALE)

    def compute_chunk(j):
        @plsc.parallel_loop(j * _POS_PER_CHUNK, (j + 1) * _POS_PER_CHUNK)
        def _(p):
            for c in range(_CPR):
                sl = pl.ds(c * _LANES, _LANES)
                pe_c = pe_v[p, sl]
                for b in range(BATCH):
                    r = p * BATCH + b
                    rows_v[r, sl] = rows_v[r, sl] * scale + pe_c

    pe_copy.wait()
    stores = []
    for j in range(_N_CHUNKS):
        gathers[j].wait()
        compute_chunk(j)
        stores.append(
            pltpu.async_copy(
                rows_v.at[pl.ds(j * _IDX_CHUNK, _IDX_CHUNK)],
                out_hbm.at[pl.ds(base + j * _IDX_CHUNK, _IDX_CHUNK)],
                sems[j],
            )
        )
    for s in stores:
        s.wait()


_sc_embed = functools.partial(
    pl.kernel,
    out_type=jax.ShapeDtypeStruct((N_TOK, D_MODEL), jnp.float32),
    mesh=plsc.VectorSubcoreMesh(core_axis_name="c", subcore_axis_name="s"),
    scratch_types=[
        pltpu.VMEM((_N_CHUNKS, _IDX_CHUNK), jnp.int32),
        pltpu.VMEM((_ROWS_PER_W, D_MODEL), jnp.float32),
        pltpu.VMEM((_POS_PER_W, D_MODEL), jnp.float32),
        [pltpu.SemaphoreType.DMA] * _N_CHUNKS,
        pltpu.SemaphoreType.DMA,
    ],
)(_sc_body)


@jax.jit
def kernel(x, table, pos_enc):
    x_flat = jnp.reshape(x.astype(jnp.int32), (_NW * _N_CHUNKS, _IDX_CHUNK))
    pe = jnp.reshape(pos_enc, (pos_enc.shape[0], D_MODEL))
    out = _sc_embed(x_flat, pe, table)
    return jnp.reshape(out, (SEQ_LEN, BATCH, D_MODEL))
